# baseline (speedup 1.0000x reference)
import functools

import jax
import jax.numpy as jnp
from jax import lax
from jax.experimental import pallas as pl
from jax.experimental.pallas import tpu as pltpu
from jax.experimental.pallas import tpu_sc as plsc

_N = 10000
_E = 320000
_HID = 128
_OUT = 64
_LAMBDA = 0.1
_ALPHA = 0.5
_EPS = 1e-5

_NTILES = 32
_CH = 64
_NCHUNK = 160
_EPAD = _NTILES * _NCHUNK * _CH
_NACC = 10240
_RPT = _NACC // 16
_GSZ = 8
_NGRP = _NCHUNK // _GSZ
_NBUF = 4


@functools.cache
def _get_sc_deg():
    mesh = plsc.VectorSubcoreMesh(core_axis_name="c", subcore_axis_name="s")
    return functools.partial(
        pl.kernel,
        mesh=mesh,
        out_type=jax.ShapeDtypeStruct((2, _NACC, _HID), jnp.float32),
        scratch_types=[
            pltpu.VMEM((_NCHUNK, _CH), jnp.int32),
            pltpu.VMEM((_CH, _HID), jnp.float32),
            pltpu.VMEM_SHARED((_NACC, _HID), jnp.float32),
        ],
    )(_sc_deg_body)


def _sc_deg_body(dst_hbm, out_hbm, dst_v, ones_v, acc_sh):
    cid = lax.axis_index("c")
    sid = lax.axis_index("s")
    wid = cid * 16 + sid

    def _zero(r, carry):
        for c in range(_HID // 16):
            ones_v[r, pl.ds(c * 16, 16)] = jnp.zeros((16,), jnp.float32)
        return carry
    lax.fori_loop(0, _CH, _zero, 0)
    base = sid * _RPT
    for k in range(_RPT // _CH):
        pltpu.sync_copy(ones_v, acc_sh.at[pl.ds(base + k * _CH, _CH)])
    if _RPT % _CH:
        pltpu.sync_copy(ones_v.at[pl.ds(0, _RPT % _CH)],
                        acc_sh.at[pl.ds(base + (_RPT // _CH) * _CH, _RPT % _CH)])

    def _fill(r, carry):
        for c in range(_HID // 16):
            ones_v[r, pl.ds(c * 16, 16)] = jnp.ones((16,), jnp.float32)
        return carry
    lax.fori_loop(0, _CH, _fill, 0)
    plsc.subcore_barrier()

    pltpu.sync_copy(dst_hbm.at[wid], dst_v)

    def _body(j, carry):
        pltpu.sync_copy(ones_v, acc_sh.at[dst_v.at[j]], add=True)
        return carry
    lax.fori_loop(0, _NCHUNK, _body, 0)
    plsc.subcore_barrier()

    pltpu.sync_copy(acc_sh.at[pl.ds(base, _RPT)], out_hbm.at[cid, pl.ds(base, _RPT)])


@functools.cache
def _get_sc_agg():
    mesh = plsc.VectorSubcoreMesh(core_axis_name="c", subcore_axis_name="s")
    return functools.partial(
        pl.kernel,
        mesh=mesh,
        out_type=jax.ShapeDtypeStruct((2, _NACC, _HID), jnp.float32),
        scratch_types=[
            pltpu.VMEM((2, _GSZ, _CH), jnp.int32),
            pltpu.VMEM((2, _GSZ, _CH), jnp.int32),
            pltpu.VMEM((_CH, _HID), jnp.float32),
            pltpu.VMEM((_CH, _HID), jnp.float32),
            pltpu.VMEM((_CH, _HID), jnp.float32),
            pltpu.VMEM((_CH, _HID), jnp.float32),
            pltpu.VMEM_SHARED((_NACC, _HID), jnp.float32),
            pltpu.SemaphoreType.DMA,
            pltpu.SemaphoreType.DMA,
            pltpu.SemaphoreType.DMA,
            pltpu.SemaphoreType.DMA,
            pltpu.SemaphoreType.DMA,
        ],
    )(_sc_agg_body)


def _sc_agg_body(h_hbm, src_hbm, dst_hbm, out_hbm, srcst, dstst, rows0, rows1,
                 rows2, rows3, acc_sh, sem0, sem1, sem2, sem3, sem_idx):
    cid = lax.axis_index("c")
    sid = lax.axis_index("s")
    wid = cid * 16 + sid
    bufs = (rows0, rows1, rows2, rows3)
    sems = (sem0, sem1, sem2, sem3)

    def _zero(r, carry):
        for c in range(_HID // 16):
            rows0[r, pl.ds(c * 16, 16)] = jnp.zeros((16,), jnp.float32)
        return carry
    lax.fori_loop(0, _CH, _zero, 0)
    base = sid * _RPT
    for k in range(_RPT // _CH):
        pltpu.sync_copy(rows0, acc_sh.at[pl.ds(base + k * _CH, _CH)])
    if _RPT % _CH:
        pltpu.sync_copy(rows0.at[pl.ds(0, _RPT % _CH)],
                        acc_sh.at[pl.ds(base + (_RPT // _CH) * _CH, _RPT % _CH)])
    plsc.subcore_barrier()

    pltpu.sync_copy(src_hbm.at[wid, pl.ds(0, _GSZ)], srcst.at[0])
    pltpu.sync_copy(dst_hbm.at[wid, pl.ds(0, _GSZ)], dstst.at[0])
    for b in range(_NBUF):
        pltpu.async_copy(h_hbm.at[srcst.at[0, b]], bufs[b], sems[b])

    def _group(g, carry):
        pg = lax.rem(g, 2)
        ng = lax.rem(g + 1, 2)
        gn = lax.rem(g + 1, _NGRP)
        pltpu.async_copy(src_hbm.at[wid, pl.ds(gn * _GSZ, _GSZ)],
                         srcst.at[ng], sem_idx)
        pltpu.async_copy(dst_hbm.at[wid, pl.ds(gn * _GSZ, _GSZ)],
                         dstst.at[ng], sem_idx)
        for b in range(_GSZ):
            buf, sem = bufs[b % _NBUF], sems[b % _NBUF]
            if b == _GSZ - _NBUF - 1:
                pltpu.make_async_copy(src_hbm.at[wid, pl.ds(gn * _GSZ, _GSZ)],
                                      srcst.at[ng], sem_idx).wait()
                pltpu.make_async_copy(dst_hbm.at[wid, pl.ds(gn * _GSZ, _GSZ)],
                                      dstst.at[ng], sem_idx).wait()
            pltpu.make_async_copy(h_hbm.at[srcst.at[pg, b]], buf, sem).wait()
            pltpu.sync_copy(buf, acc_sh.at[dstst.at[pg, b]], add=True)
            if b < _GSZ - _NBUF:
                pltpu.async_copy(h_hbm.at[srcst.at[pg, b + _NBUF]], buf, sem)
            else:
                pltpu.async_copy(h_hbm.at[srcst.at[ng, b - (_GSZ - _NBUF)]], buf, sem)
        return carry
    lax.fori_loop(0, _NGRP, _group, 0)
    for b in range(_NBUF):
        pltpu.make_async_copy(h_hbm.at[srcst.at[0, b]], bufs[b], sems[b]).wait()
    plsc.subcore_barrier()

    pltpu.sync_copy(acc_sh.at[pl.ds(base, _RPT)], out_hbm.at[cid, pl.ds(base, _RPT)])


_RB = 2000


def _mlp_body(x_ref, w_ref, b_ref, o_ref):
    o_ref[...] = jnp.maximum(
        jnp.dot(x_ref[...], w_ref[...], preferred_element_type=jnp.float32)
        + b_ref[...], 0.0)


def _tc_mlp(x, W, b2):
    return pl.pallas_call(
        _mlp_body,
        grid=(_N // _RB,),
        in_specs=[
            pl.BlockSpec((_RB, _HID), lambda i: (i, 0)),
            pl.BlockSpec((_HID, _HID), lambda i: (0, 0)),
            pl.BlockSpec((1, _HID), lambda i: (0, 0)),
        ],
        out_specs=pl.BlockSpec((_RB, _HID), lambda i: (i, 0)),
        out_shape=jax.ShapeDtypeStruct((_N, _HID), jnp.float32),
    )(x, W, b2)


def _attn_body(h_ref, wq_ref, kt_ref, v_ref, o_ref):
    q = jnp.dot(h_ref[...], wq_ref[...], preferred_element_type=jnp.float32)
    s = jnp.dot(q, kt_ref[...], preferred_element_type=jnp.float32)
    m = jnp.max(s, axis=-1, keepdims=True)
    e = jnp.exp(s - m)
    attn = e / jnp.sum(e, axis=-1, keepdims=True)
    o_ref[...] = jnp.dot(attn, v_ref[...], preferred_element_type=jnp.float32)


def _tc_attn(h, Wq, kT_beta, Vmem):
    return pl.pallas_call(
        _attn_body,
        grid=(_N // _RB,),
        in_specs=[
            pl.BlockSpec((_RB, _HID), lambda i: (i, 0)),
            pl.BlockSpec((_HID, _HID), lambda i: (0, 0)),
            pl.BlockSpec((_HID, 32), lambda i: (0, 0)),
            pl.BlockSpec((32, _HID), lambda i: (0, 0)),
        ],
        out_specs=pl.BlockSpec((_RB, _HID), lambda i: (i, 0)),
        out_shape=jax.ShapeDtypeStruct((_N, _HID), jnp.float32),
    )(h, Wq, kT_beta, Vmem)


def _new_h(h_ref, r_ref, a_ref, d_ref, g_ref, bl_ref):
    aggs = a_ref[0] + a_ref[1]
    degs = d_ref[0] + d_ref[1]
    deg = jnp.maximum(degs[:, 0:1], 1.0)
    h_new = r_ref[...] + (_LAMBDA / deg) * aggs
    h = (1.0 - _ALPHA) * h_ref[...] + _ALPHA * h_new
    mu = jnp.mean(h, axis=-1, keepdims=True)
    var = jnp.mean((h - mu) * (h - mu), axis=-1, keepdims=True)
    return g_ref[...] * (h - mu) / jnp.sqrt(var + _EPS) + bl_ref[...]


def _combine_body(h_ref, r_ref, a_ref, d_ref, g_ref, bl_ref, o_ref):
    o_ref[...] = _new_h(h_ref, r_ref, a_ref, d_ref, g_ref, bl_ref)


def _combine_cls_body(h_ref, r_ref, a_ref, d_ref, g_ref, bl_ref, wc_ref,
                      bc_ref, o_ref):
    hn = _new_h(h_ref, r_ref, a_ref, d_ref, g_ref, bl_ref)
    o_ref[...] = (jnp.dot(hn, wc_ref[...], preferred_element_type=jnp.float32)
                  + bc_ref[...])


_common_combine_specs = [
    pl.BlockSpec((_RB, _HID), lambda i: (i, 0)),
    pl.BlockSpec((_RB, _HID), lambda i: (i, 0)),
    pl.BlockSpec((2, _RB, _HID), lambda i: (0, i, 0)),
    pl.BlockSpec((2, _RB, _HID), lambda i: (0, i, 0)),
    pl.BlockSpec((1, _HID), lambda i: (0, 0)),
    pl.BlockSpec((1, _HID), lambda i: (0, 0)),
]


def _tc_combine(h, retr, aggp, degp, g2, bl2):
    return pl.pallas_call(
        _combine_body,
        grid=(_N // _RB,),
        in_specs=list(_common_combine_specs),
        out_specs=pl.BlockSpec((_RB, _HID), lambda i: (i, 0)),
        out_shape=jax.ShapeDtypeStruct((_N, _HID), jnp.float32),
    )(h, retr, aggp, degp, g2, bl2)


def _tc_combine_cls(h, retr, aggp, degp, g2, bl2, Wc, bc2):
    return pl.pallas_call(
        _combine_cls_body,
        grid=(_N // _RB,),
        in_specs=list(_common_combine_specs) + [
            pl.BlockSpec((_HID, _OUT), lambda i: (0, 0)),
            pl.BlockSpec((1, _OUT), lambda i: (0, 0)),
        ],
        out_specs=pl.BlockSpec((_RB, _OUT), lambda i: (i, 0)),
        out_shape=jax.ShapeDtypeStruct((_N, _OUT), jnp.float32),
    )(h, retr, aggp, degp, g2, bl2, Wc, bc2)


def kernel(x, edge_index, W_in, b_in, Wq, Kmem, Vmem, beta_param, gamma,
           beta_ln, Wc, bc):
    src = edge_index[0]
    dst = edge_index[1]
    pad = _EPAD - _E
    src_p = jnp.concatenate(
        [src, jnp.zeros((pad,), jnp.int32)]).reshape(_NTILES, _NCHUNK, _CH)
    dst_p = jnp.concatenate(
        [dst, jnp.full((pad,), _N, jnp.int32)]).reshape(_NTILES, _NCHUNK, _CH)

    kT_beta = beta_param * Kmem.T
    b2 = b_in.reshape(1, _HID)
    g2 = gamma.reshape(1, _HID)
    bl2 = beta_ln.reshape(1, _HID)
    bc2 = bc.reshape(1, _OUT)

    degp = _get_sc_deg()(dst_p)
    h = _tc_mlp(x, W_in, b2)
    logits = None
    for t in range(2):
        aggp = _get_sc_agg()(h, src_p, dst_p)
        retr = _tc_attn(h, Wq, kT_beta, Vmem)
        if t == 0:
            h = _tc_combine(h, retr, aggp, degp, g2, bl2)
        else:
            logits = _tc_combine_cls(h, retr, aggp, degp, g2, bl2, Wc, bc2)
    return logits


# speedup vs baseline: 1.2960x; 1.2960x over previous
import functools

import jax
import jax.numpy as jnp
from jax import lax
from jax.experimental import pallas as pl
from jax.experimental.pallas import tpu as pltpu
from jax.experimental.pallas import tpu_sc as plsc

_N = 10000
_E = 320000
_HID = 128
_OUT = 64
_LAMBDA = 0.1
_ALPHA = 0.5
_EPS = 1e-5

_NTILES = 32
_CH = 64
_NCHUNK = 160
_EPAD = _NTILES * _NCHUNK * _CH
_NACC = 10240
_RPT = _NACC // 16
_GSZ = 8
_NBUF = 4
_FAST_CID = 0
_NCHF = 256
_NCHS = 64
_TOTCH = _EPAD // _CH
_DCH = 128
_DNCHUNK = _EPAD // _NTILES // _DCH


@functools.cache
def _get_sc_deg():
    mesh = plsc.VectorSubcoreMesh(core_axis_name="c", subcore_axis_name="s")
    return functools.partial(
        pl.kernel,
        mesh=mesh,
        out_type=jax.ShapeDtypeStruct((2, _NACC, _HID), jnp.float32),
        scratch_types=[
            pltpu.VMEM((_DNCHUNK, _DCH), jnp.int32),
            pltpu.VMEM((_DCH, _HID), jnp.float32),
            pltpu.VMEM_SHARED((_NACC, _HID), jnp.float32),
        ],
    )(_sc_deg_body)


def _sc_deg_body(dst_hbm, out_hbm, dst_v, ones_v, acc_sh):
    cid = lax.axis_index("c")
    sid = lax.axis_index("s")
    wid = cid * 16 + sid

    def _zero(r, carry):
        for c in range(_HID // 16):
            ones_v[r, pl.ds(c * 16, 16)] = jnp.zeros((16,), jnp.float32)
        return carry
    lax.fori_loop(0, _DCH, _zero, 0)
    base = sid * _RPT
    for k in range(_RPT // _DCH):
        pltpu.sync_copy(ones_v, acc_sh.at[pl.ds(base + k * _DCH, _DCH)])
    if _RPT % _DCH:
        pltpu.sync_copy(ones_v.at[pl.ds(0, _RPT % _DCH)],
                        acc_sh.at[pl.ds(base + (_RPT // _DCH) * _DCH, _RPT % _DCH)])

    def _fill(r, carry):
        for c in range(_HID // 16):
            ones_v[r, pl.ds(c * 16, 16)] = jnp.ones((16,), jnp.float32)
        return carry
    lax.fori_loop(0, _DCH, _fill, 0)
    plsc.subcore_barrier()

    pltpu.sync_copy(dst_hbm.at[wid], dst_v)

    def _body(j, carry):
        pltpu.sync_copy(ones_v, acc_sh.at[dst_v.at[j]], add=True)
        return carry
    lax.fori_loop(0, _DNCHUNK, _body, 0)
    plsc.subcore_barrier()

    pltpu.sync_copy(acc_sh.at[pl.ds(base, _RPT)], out_hbm.at[cid, pl.ds(base, _RPT)])


@functools.cache
def _get_sc_agg():
    mesh = plsc.VectorSubcoreMesh(core_axis_name="c", subcore_axis_name="s")
    return functools.partial(
        pl.kernel,
        mesh=mesh,
        out_type=jax.ShapeDtypeStruct((2, _NACC, _HID), jnp.float32),
        scratch_types=[
            pltpu.VMEM((2, _GSZ, _CH), jnp.int32),
            pltpu.VMEM((2, _GSZ, _CH), jnp.int32),
            pltpu.VMEM((_CH, _HID), jnp.float32),
            pltpu.VMEM((_CH, _HID), jnp.float32),
            pltpu.VMEM((_CH, _HID), jnp.float32),
            pltpu.VMEM((_CH, _HID), jnp.float32),
            pltpu.VMEM_SHARED((_NACC, _HID), jnp.float32),
            pltpu.SemaphoreType.DMA,
            pltpu.SemaphoreType.DMA,
            pltpu.SemaphoreType.DMA,
            pltpu.SemaphoreType.DMA,
            pltpu.SemaphoreType.DMA,
        ],
    )(_sc_agg_body)


def _sc_agg_body(h_hbm, src_hbm, dst_hbm, out_hbm, srcst, dstst, rows0, rows1,
                 rows2, rows3, acc_sh, sem0, sem1, sem2, sem3, sem_idx):
    cid = lax.axis_index("c")
    sid = lax.axis_index("s")
    bufs = (rows0, rows1, rows2, rows3)
    sems = (sem0, sem1, sem2, sem3)

    on_fast = cid == _FAST_CID
    ngrp = jnp.where(on_fast, _NCHF // _GSZ, _NCHS // _GSZ)
    cbase = pl.multiple_of(
        jnp.where(on_fast, sid * _NCHF, 16 * _NCHF + sid * _NCHS), _GSZ)

    def _zero(r, carry):
        for c in range(_HID // 16):
            rows0[r, pl.ds(c * 16, 16)] = jnp.zeros((16,), jnp.float32)
        return carry
    lax.fori_loop(0, _CH, _zero, 0)
    base = sid * _RPT
    for k in range(_RPT // _CH):
        pltpu.sync_copy(rows0, acc_sh.at[pl.ds(base + k * _CH, _CH)])
    plsc.subcore_barrier()

    pltpu.sync_copy(src_hbm.at[pl.ds(cbase, _GSZ)], srcst.at[0])
    pltpu.sync_copy(dst_hbm.at[pl.ds(cbase, _GSZ)], dstst.at[0])
    for b in range(_NBUF):
        pltpu.async_copy(h_hbm.at[srcst.at[0, b]], bufs[b], sems[b])

    def _group(g, carry):
        pg = lax.rem(g, 2)
        ng = lax.rem(g + 1, 2)
        gn = lax.rem(g + 1, ngrp)
        goff = pl.multiple_of(cbase + gn * _GSZ, _GSZ)
        pltpu.async_copy(src_hbm.at[pl.ds(goff, _GSZ)], srcst.at[ng], sem_idx)
        pltpu.async_copy(dst_hbm.at[pl.ds(goff, _GSZ)], dstst.at[ng], sem_idx)
        for b in range(_GSZ):
            buf, sem = bufs[b % _NBUF], sems[b % _NBUF]
            if b == _GSZ - _NBUF - 1:
                pltpu.make_async_copy(src_hbm.at[pl.ds(goff, _GSZ)],
                                      srcst.at[ng], sem_idx).wait()
                pltpu.make_async_copy(dst_hbm.at[pl.ds(goff, _GSZ)],
                                      dstst.at[ng], sem_idx).wait()
            pltpu.make_async_copy(h_hbm.at[srcst.at[pg, b]], buf, sem).wait()
            pltpu.sync_copy(buf, acc_sh.at[dstst.at[pg, b]], add=True)
            if b < _GSZ - _NBUF:
                pltpu.async_copy(h_hbm.at[srcst.at[pg, b + _NBUF]], buf, sem)
            else:
                pltpu.async_copy(h_hbm.at[srcst.at[ng, b - (_GSZ - _NBUF)]], buf, sem)
        return carry
    lax.fori_loop(0, ngrp, _group, 0)
    for b in range(_NBUF):
        pltpu.make_async_copy(h_hbm.at[srcst.at[0, b]], bufs[b], sems[b]).wait()
    plsc.subcore_barrier()

    pltpu.sync_copy(acc_sh.at[pl.ds(base, _RPT)], out_hbm.at[cid, pl.ds(base, _RPT)])


_RB = 2000


def _mlp_body(x_ref, w_ref, b_ref, o_ref):
    o_ref[...] = jnp.maximum(
        jnp.dot(x_ref[...], w_ref[...], preferred_element_type=jnp.float32)
        + b_ref[...], 0.0)


def _tc_mlp(x, W, b2):
    return pl.pallas_call(
        _mlp_body,
        grid=(_N // _RB,),
        in_specs=[
            pl.BlockSpec((_RB, _HID), lambda i: (i, 0)),
            pl.BlockSpec((_HID, _HID), lambda i: (0, 0)),
            pl.BlockSpec((1, _HID), lambda i: (0, 0)),
        ],
        out_specs=pl.BlockSpec((_RB, _HID), lambda i: (i, 0)),
        out_shape=jax.ShapeDtypeStruct((_N, _HID), jnp.float32),
    )(x, W, b2)


def _attn_body(h_ref, wq_ref, kt_ref, v_ref, o_ref):
    q = jnp.dot(h_ref[...], wq_ref[...], preferred_element_type=jnp.float32)
    s = jnp.dot(q, kt_ref[...], preferred_element_type=jnp.float32)
    m = jnp.max(s, axis=-1, keepdims=True)
    e = jnp.exp(s - m)
    attn = e / jnp.sum(e, axis=-1, keepdims=True)
    o_ref[...] = jnp.dot(attn, v_ref[...], preferred_element_type=jnp.float32)


def _tc_attn(h, Wq, kT_beta, Vmem):
    return pl.pallas_call(
        _attn_body,
        grid=(_N // _RB,),
        in_specs=[
            pl.BlockSpec((_RB, _HID), lambda i: (i, 0)),
            pl.BlockSpec((_HID, _HID), lambda i: (0, 0)),
            pl.BlockSpec((_HID, 32), lambda i: (0, 0)),
            pl.BlockSpec((32, _HID), lambda i: (0, 0)),
        ],
        out_specs=pl.BlockSpec((_RB, _HID), lambda i: (i, 0)),
        out_shape=jax.ShapeDtypeStruct((_N, _HID), jnp.float32),
    )(h, Wq, kT_beta, Vmem)


def _new_h(h_ref, r_ref, a_ref, d_ref, g_ref, bl_ref):
    aggs = a_ref[0] + a_ref[1]
    degs = d_ref[0] + d_ref[1]
    deg = jnp.maximum(degs[:, 0:1], 1.0)
    h_new = r_ref[...] + (_LAMBDA / deg) * aggs
    h = (1.0 - _ALPHA) * h_ref[...] + _ALPHA * h_new
    mu = jnp.mean(h, axis=-1, keepdims=True)
    var = jnp.mean((h - mu) * (h - mu), axis=-1, keepdims=True)
    return g_ref[...] * (h - mu) / jnp.sqrt(var + _EPS) + bl_ref[...]


def _combine_body(h_ref, r_ref, a_ref, d_ref, g_ref, bl_ref, o_ref):
    o_ref[...] = _new_h(h_ref, r_ref, a_ref, d_ref, g_ref, bl_ref)


def _combine_cls_body(h_ref, r_ref, a_ref, d_ref, g_ref, bl_ref, wc_ref,
                      bc_ref, o_ref):
    hn = _new_h(h_ref, r_ref, a_ref, d_ref, g_ref, bl_ref)
    o_ref[...] = (jnp.dot(hn, wc_ref[...], preferred_element_type=jnp.float32)
                  + bc_ref[...])


_common_combine_specs = [
    pl.BlockSpec((_RB, _HID), lambda i: (i, 0)),
    pl.BlockSpec((_RB, _HID), lambda i: (i, 0)),
    pl.BlockSpec((2, _RB, _HID), lambda i: (0, i, 0)),
    pl.BlockSpec((2, _RB, _HID), lambda i: (0, i, 0)),
    pl.BlockSpec((1, _HID), lambda i: (0, 0)),
    pl.BlockSpec((1, _HID), lambda i: (0, 0)),
]


def _tc_combine(h, retr, aggp, degp, g2, bl2):
    return pl.pallas_call(
        _combine_body,
        grid=(_N // _RB,),
        in_specs=list(_common_combine_specs),
        out_specs=pl.BlockSpec((_RB, _HID), lambda i: (i, 0)),
        out_shape=jax.ShapeDtypeStruct((_N, _HID), jnp.float32),
    )(h, retr, aggp, degp, g2, bl2)


def _tc_combine_cls(h, retr, aggp, degp, g2, bl2, Wc, bc2):
    return pl.pallas_call(
        _combine_cls_body,
        grid=(_N // _RB,),
        in_specs=list(_common_combine_specs) + [
            pl.BlockSpec((_HID, _OUT), lambda i: (0, 0)),
            pl.BlockSpec((1, _OUT), lambda i: (0, 0)),
        ],
        out_specs=pl.BlockSpec((_RB, _OUT), lambda i: (i, 0)),
        out_shape=jax.ShapeDtypeStruct((_N, _OUT), jnp.float32),
    )(h, retr, aggp, degp, g2, bl2, Wc, bc2)


def kernel(x, edge_index, W_in, b_in, Wq, Kmem, Vmem, beta_param, gamma,
           beta_ln, Wc, bc):
    src = edge_index[0]
    dst = edge_index[1]
    pad = _EPAD - _E
    src_p = jnp.concatenate(
        [src, jnp.zeros((pad,), jnp.int32)]).reshape(_TOTCH, _CH)
    dst_flat = jnp.concatenate([dst, jnp.full((pad,), _N, jnp.int32)])
    dst_p = dst_flat.reshape(_TOTCH, _CH)
    dst_pd = dst_flat.reshape(_NTILES, _DNCHUNK, _DCH)

    kT_beta = beta_param * Kmem.T
    b2 = b_in.reshape(1, _HID)
    g2 = gamma.reshape(1, _HID)
    bl2 = beta_ln.reshape(1, _HID)
    bc2 = bc.reshape(1, _OUT)

    degp = _get_sc_deg()(dst_pd)
    h = _tc_mlp(x, W_in, b2)
    logits = None
    for t in range(2):
        aggp = _get_sc_agg()(h, src_p, dst_p)
        retr = _tc_attn(h, Wq, kT_beta, Vmem)
        if t == 0:
            h = _tc_combine(h, retr, aggp, degp, g2, bl2)
        else:
            logits = _tc_combine_cls(h, retr, aggp, degp, g2, bl2, Wc, bc2)
    return logits
